# Initial kernel scaffold; baseline (speedup 1.0000x reference)
import jax
import jax.numpy as jnp
from jax.experimental import pallas as pl


def kernel(x, edge_index, batch, W1, b1, W2, b2, W3, b3, fc1_W, fc1_b, fc2_W, fc2_b):
    raise NotImplementedError("write your pallas kernel here")


# speedup vs baseline: 2.3094x; 2.3094x over previous
import jax
import jax.numpy as jnp
from jax.experimental import pallas as pl

NUM_GRAPHS = 1024


def _head_body(g_ref, W3_ref, b3_ref, f1W_ref, f1b_ref, f2W_ref, f2b_ref, out_ref):
    g = g_ref[...]
    G = jnp.dot(g, W3_ref[...], preferred_element_type=jnp.float32) + b3_ref[...]
    G = jnp.maximum(jnp.dot(G, f1W_ref[...], preferred_element_type=jnp.float32) + f1b_ref[...], 0.0)
    out_ref[...] = jnp.dot(G, f2W_ref[...], preferred_element_type=jnp.float32) + f2b_ref[...]


def kernel(x, edge_index, batch, W1, b1, W2, b2, W3, b3, fc1_W, fc1_b, fc2_W, fc2_b):
    n = x.shape[0]
    src = edge_index[0].astype(jnp.int32)
    dst = edge_index[1].astype(jnp.int32)
    batch = batch.astype(jnp.int32)

    deg = jnp.zeros((n,), jnp.float32).at[dst].add(1.0) + 1.0
    dinv = jax.lax.rsqrt(deg)

    def agg(h):
        t = dinv[:, None] * h
        s = jnp.zeros_like(t).at[dst].add(t[src])
        return dinv[:, None] * (s + t)

    a0 = agg(x)
    h1 = jax.nn.relu(a0 * W1[0][None, :] + b1[None, :])
    a1 = agg(h1)
    h2 = jax.nn.relu(a1 @ W2 + b2[None, :])
    a2 = agg(h2)

    sums = jnp.zeros((NUM_GRAPHS, a2.shape[1]), jnp.float32).at[batch].add(a2)
    cnt = jnp.zeros((NUM_GRAPHS,), jnp.float32).at[batch].add(1.0)
    g = sums / jnp.maximum(cnt, 1.0)[:, None]

    f2Wp = jnp.zeros((128, 128), jnp.float32).at[:, :12].set(fc2_W)
    f2bp = jnp.zeros((128,), jnp.float32).at[:12].set(fc2_b)

    out_p = pl.pallas_call(
        _head_body,
        out_shape=jax.ShapeDtypeStruct((NUM_GRAPHS, 128), jnp.float32),
    )(g, W3, b3[None, :], fc1_W, fc1_b[None, :], f2Wp, f2bp[None, :])
    return out_p[:, :12]


# speedup vs baseline: 5.8993x; 2.5544x over previous
import functools

import jax
import jax.numpy as jnp
from jax import lax
from jax.experimental import pallas as pl
from jax.experimental.pallas import tpu as pltpu
from jax.experimental.pallas import tpu_sc as plsc

N = 50000
NE = 800000
NG = 1024

NPAD = 53248
NROWS = NPAD // 128
NE_P = 819200
EROWS = NE_P // 128
POOL_PAD = 1152
NC, NS = 2, 16
ROWS_T = NPAD // NS
CROWS = 72
QR = 8336
NQ = 6
ACC_R = 8448
WR_F = 528

_f32 = jnp.float32


def _ds(ref, start, size):
    return ref.at[pl.ds(start, size)]


@functools.lru_cache(maxsize=None)
def _build_sc_kernels():
    mesh = plsc.VectorSubcoreMesh(core_axis_name="c", subcore_axis_name="s")

    G_H = 8
    @functools.partial(
        pl.kernel,
        mesh=mesh,
        out_type=[
            jax.ShapeDtypeStruct((NC, NPAD), _f32),
            jax.ShapeDtypeStruct((NC, POOL_PAD), _f32),
        ],
        scratch_types=[
            pltpu.VMEM_SHARED((NPAD,), _f32),
            pltpu.VMEM_SHARED((POOL_PAD,), _f32),
            pltpu.VMEM((16, 128), _f32),
            pltpu.VMEM((G_H, 128), jnp.int32),
            pltpu.VMEM((G_H, 128), jnp.int32),
            pltpu.VMEM((16, 128), jnp.int32),
            pltpu.SemaphoreType.DMA,
        ],
    )
    def sc_hist(dstE, batchE, ones_hbm, z1, zc, degp, cntp,
                acc_deg, acc_cnt, ones_v, idx0, idx1, idxb_v, sem_s):
        c = lax.axis_index("c")
        s = lax.axis_index("s")
        w = s * NC + c
        idxs = [idx0, idx1]
        pltpu.sync_copy(_ds(z1, s * ROWS_T, ROWS_T), _ds(acc_deg, s * ROWS_T, ROWS_T))

        @pl.when(s < POOL_PAD // 128)
        def _():
            pltpu.sync_copy(_ds(zc, s * 128, 128), _ds(acc_cnt, s * 128, 128))
        pltpu.sync_copy(ones_hbm, ones_v)
        plsc.subcore_barrier()

        pltpu.sync_copy(_ds(dstE, w * 200, G_H), idx0)

        def scat_group(buf):
            for j in range(G_H):
                pltpu.async_copy(ones_v.at[j], acc_deg.at[buf.at[j]],
                                 sem_s, add=True)
            for j in range(G_H):
                pltpu.make_async_copy(z1.at[pl.ds(0, 128)],
                                      ones_v.at[j], sem_s).wait()

        def outer(o, carry):
            for b in (0, 1):
                g = o * 2 + b

                @pl.when(g < 24)
                def _():
                    pltpu.sync_copy(_ds(dstE, w * 200 + (g + 1) * G_H, G_H),
                                    idxs[1 - b])
                scat_group(idxs[b])
            return carry

        lax.fori_loop(0, 12, outer, 0)
        scat_group(idx0)

        @pl.when(w < 26)
        def _():
            pltpu.sync_copy(_ds(batchE, w * 16, 16), idxb_v)
            for j in range(16):
                pltpu.sync_copy(ones_v.at[j], acc_cnt.at[idxb_v.at[j]], add=True)
        plsc.subcore_barrier()
        pltpu.sync_copy(_ds(acc_deg, s * ROWS_T, ROWS_T),
                        degp.at[c].at[pl.ds(s * ROWS_T, ROWS_T)])

        @pl.when(s < POOL_PAD // 128)
        def _():
            pltpu.sync_copy(_ds(acc_cnt, s * 128, 128),
                            cntp.at[c].at[pl.ds(s * 128, 128)])

    G_0 = 8
    @functools.partial(
        pl.kernel,
        mesh=mesh,
        out_type=jax.ShapeDtypeStruct((NC, NPAD), _f32),
        scratch_types=[
            pltpu.VMEM_SHARED((NPAD,), _f32),
            pltpu.VMEM((G_0, 128), jnp.int32),
            pltpu.VMEM((G_0, 128), jnp.int32),
            pltpu.VMEM((G_0, 128), jnp.int32),
            pltpu.VMEM((G_0, 128), jnp.int32),
            pltpu.VMEM((G_0, 128), _f32),
            pltpu.VMEM((G_0, 128), _f32),
            pltpu.SemaphoreType.DMA,
            pltpu.SemaphoreType.DMA,
        ],
    )
    def sc_s0(srcE, dstE, t0, z1, s0p, acc,
              idxs0, idxs1, idxd0, idxd1, vals0, vals1, sem_g, sem_s):
        c = lax.axis_index("c")
        s = lax.axis_index("s")
        w = s * NC + c
        idxs = [idxs0, idxs1]
        idxd = [idxd0, idxd1]
        vals = [vals0, vals1]
        pltpu.sync_copy(_ds(z1, s * ROWS_T, ROWS_T), _ds(acc, s * ROWS_T, ROWS_T))
        plsc.subcore_barrier()

        def load_idx(g, b):
            off = w * 200 + g * G_0
            pltpu.sync_copy(_ds(srcE, off, G_0), idxs[b])
            pltpu.sync_copy(_ds(dstE, off, G_0), idxd[b])

        def fire_gathers(b):
            for j in range(G_0):
                pltpu.async_copy(t0.at[idxs[b].at[j]], vals[b].at[j], sem_g)

        load_idx(0, 0)
        fire_gathers(0)

        def drain_group(b):
            for j in range(G_0):
                pltpu.make_async_copy(z1.at[pl.ds(0, 128)],
                                      vals[b].at[j], sem_g).wait()

        def scat_group(b):
            for j in range(G_0):
                pltpu.async_copy(vals[b].at[j], acc.at[idxd[b].at[j]],
                                 sem_s, add=True)
            for j in range(G_0):
                pltpu.make_async_copy(z1.at[pl.ds(0, 128)],
                                      vals[b].at[j], sem_s).wait()

        def outer(o, carry):
            for b in (0, 1):
                g = o * 2 + b
                drain_group(b)

                @pl.when(g < 24)
                def _():
                    load_idx(g + 1, 1 - b)
                    fire_gathers(1 - b)
                scat_group(b)
            return carry

        lax.fori_loop(0, 12, outer, 0)
        drain_group(0)
        scat_group(0)
        plsc.subcore_barrier()
        pltpu.sync_copy(_ds(acc, s * ROWS_T, ROWS_T),
                        s0p.at[c].at[pl.ds(s * ROWS_T, ROWS_T)])

    def make_sc_agg():
        @functools.partial(
            pl.kernel,
            mesh=mesh,
            out_type=jax.ShapeDtypeStruct((NPAD, 128), _f32),
            scratch_types=[
                pltpu.VMEM_SHARED((ACC_R, 128), _f32),
                pltpu.VMEM((8, 128), jnp.int32),
                pltpu.VMEM((8, 128), jnp.int32),
                pltpu.VMEM((128, 128), _f32),
                pltpu.VMEM((128, 128), _f32),
                pltpu.SemaphoreType.DMA,
                pltpu.SemaphoreType.DMA,
            ],
        )
        def sc_agg(srcE, idxtE, tab, z128, out,
                   acc, idxs, idxd, vals0, vals1, sem_g, sem_s):
            c = lax.axis_index("c")
            s = lax.axis_index("s")
            vals = [vals0, vals1]

            for p in range(NQ // NC):
                pltpu.sync_copy(_ds(z128, s * (ACC_R // NS), ACC_R // NS),
                                _ds(acc, s * (ACC_R // NS), ACC_R // NS))
                plsc.subcore_barrier()

                for ci in range(NC):
                    @pl.when(c == ci)
                    def _(_ci=ci, _p=p):
                        q = _ci * (NQ // NC) + _p

                        def body(g, carry):
                            off = s * 400 + g * 8
                            pltpu.sync_copy(_ds(srcE, off, 8), idxs)
                            pltpu.sync_copy(
                                _ds(idxtE, q * EROWS + off, 8), idxd)
                            pltpu.async_copy(tab.at[idxs.at[0]], vals0, sem_g)
                            for j in range(8):
                                b = j & 1
                                pltpu.make_async_copy(tab.at[pl.ds(0, 128)],
                                                      vals[b], sem_g).wait()
                                if j < 7:
                                    if j >= 1:
                                        pltpu.make_async_copy(
                                            tab.at[pl.ds(0, 128)],
                                            vals[1 - b], sem_s).wait()
                                    pltpu.async_copy(tab.at[idxs.at[j + 1]],
                                                     vals[1 - b], sem_g)
                                pltpu.async_copy(vals[b], acc.at[idxd.at[j]],
                                                 sem_s, add=True)
                            for b in (0, 1):
                                pltpu.make_async_copy(tab.at[pl.ds(0, 128)],
                                                      vals[b], sem_s).wait()
                            return carry

                        lax.fori_loop(0, 50, body, 0)
                plsc.subcore_barrier()
                for ci in range(NC):
                    @pl.when(c == ci)
                    def _(_ci=ci, _p=p):
                        q = _ci * (NQ // NC) + _p

                        @pl.when(s < NS - 1)
                        def _():
                            pltpu.sync_copy(
                                _ds(acc, s * WR_F, WR_F),
                                out.at[pl.ds(q * QR + s * WR_F, WR_F)])

                        @pl.when(s == NS - 1)
                        def _():
                            pltpu.sync_copy(
                                _ds(acc, (NS - 1) * WR_F, QR - (NS - 1) * WR_F),
                                out.at[pl.ds(q * QR + (NS - 1) * WR_F,
                                             QR - (NS - 1) * WR_F)])
                plsc.subcore_barrier()

        return sc_agg

    @functools.partial(
        pl.kernel,
        mesh=mesh,
        out_type=jax.ShapeDtypeStruct((NC, POOL_PAD, 128), _f32),
        scratch_types=[
            pltpu.VMEM_SHARED((POOL_PAD, 128), _f32),
            pltpu.VMEM((16, 128), jnp.int32),
            pltpu.VMEM((128, 128), _f32),
            pltpu.VMEM((128, 128), _f32),
            pltpu.SemaphoreType.DMA,
        ],
    )
    def sc_pool(a2, batchE, zpool, poolp, acc, idx_v, vals0, vals1, sem_s):
        c = lax.axis_index("c")
        s = lax.axis_index("s")
        w = s * NC + c
        vals = [vals0, vals1]
        pltpu.sync_copy(_ds(zpool, s * CROWS, CROWS), _ds(acc, s * CROWS, CROWS))
        plsc.subcore_barrier()

        @pl.when(w < 26)
        def _():
            pltpu.sync_copy(_ds(batchE, w * 16, 16), idx_v)
            for k in range(16):
                b = k & 1
                if k >= 2:
                    pltpu.make_async_copy(_ds(a2, 0, 128), vals[b], sem_s).wait()
                pltpu.sync_copy(_ds(a2, w * 2048 + k * 128, 128), vals[b])
                pltpu.async_copy(vals[b], acc.at[idx_v.at[k]], sem_s, add=True)
            for k in (14, 15):
                pltpu.make_async_copy(_ds(a2, 0, 128), vals[k & 1], sem_s).wait()
        plsc.subcore_barrier()
        pltpu.sync_copy(_ds(acc, s * CROWS, CROWS),
                        poolp.at[c].at[pl.ds(s * CROWS, CROWS)])

    return sc_hist, sc_s0, make_sc_agg(), sc_pool


def _prep_body(degp_ref, x_ref, mask_ref, dinv_ref, t0_ref):
    deg = degp_ref[0] + degp_ref[1] + 1.0
    dinv = lax.rsqrt(deg)
    dinv_ref[...] = dinv
    t0_ref[...] = dinv * x_ref[...] * mask_ref[...]


def _l1_body(s0p_ref, t0_ref, dinv_ref, mask_ref, W1_ref, b1_ref, t1_ref):
    s0 = s0p_ref[0] + s0p_ref[1]
    dinv = dinv_ref[...]
    a0 = dinv * (s0 + t0_ref[...])
    h1 = jnp.maximum(a0 * W1_ref[...] + b1_ref[...], 0.0)
    t1_ref[...] = dinv * mask_ref[...] * h1


def _l2_body(s1_ref, t1_ref, dinv_ref, mask_ref, W2_ref, b2_ref, t2_ref):
    dinv = dinv_ref[...]
    a1 = dinv * (s1_ref[...] + t1_ref[...])
    h2 = jnp.dot(a1, W2_ref[...], preferred_element_type=_f32)
    h2 = jnp.maximum(h2 + b2_ref[...], 0.0)
    t2_ref[...] = dinv * mask_ref[...] * h2


def _l3_body(s2_ref, t2_ref, dinv_ref, a2_ref):
    dinv = dinv_ref[...]
    a2_ref[...] = dinv * (s2_ref[...] + t2_ref[...])


def _idxt_body(dstE_ref, idxt_ref):
    d = dstE_ref[...]
    for q in range(NQ):
        lo = q * QR
        loc = jnp.where((d >= lo) & (d < lo + QR), d - lo, QR)
        idxt_ref[q, :, :] = loc


def _head_body(poolp_ref, cntp_ref, W3_ref, b3_ref, f1W_ref, f1b_ref,
               f2W_ref, f2b_ref, out_ref):
    pool = poolp_ref[0] + poolp_ref[1]
    cnt = cntp_ref[0] + cntp_ref[1]
    g = pool / jnp.maximum(cnt, 1.0)
    G = jnp.dot(g, W3_ref[...], preferred_element_type=_f32) + b3_ref[...]
    G = jnp.maximum(jnp.dot(G, f1W_ref[...], preferred_element_type=_f32)
                    + f1b_ref[...], 0.0)
    out_ref[...] = jnp.dot(G, f2W_ref[...], preferred_element_type=_f32) + f2b_ref[...]


def _node_spec(blk, ncols):
    return pl.BlockSpec((blk, ncols), lambda i: (i, 0))


def kernel(x, edge_index, batch, W1, b1, W2, b2, W3, b3, fc1_W, fc1_b, fc2_W, fc2_b):
    sc_hist, sc_s0, sc_agg, sc_pool = _build_sc_kernels()
    src = edge_index[0].astype(jnp.int32)
    dst = edge_index[1].astype(jnp.int32)
    batch = batch.astype(jnp.int32)

    epad = 50000 + (jnp.arange(NE_P - NE, dtype=jnp.int32) % (NPAD - N))
    srcE = jnp.concatenate([src, epad]).reshape(EROWS, 128)
    dstE = jnp.concatenate([dst, epad]).reshape(EROWS, 128)
    bpad = NG + (jnp.arange(NPAD - N, dtype=jnp.int32) % 32)
    batchE = jnp.concatenate([batch, bpad]).reshape(NROWS, 128)
    x_flat = jnp.concatenate([x[:, 0], jnp.zeros((NPAD - N,), _f32)])
    mask_flat = (jnp.arange(NPAD) < N).astype(_f32)

    ones_hbm = jnp.ones((16, 128), _f32)
    z1 = jnp.zeros((NPAD,), _f32)
    zc = jnp.zeros((POOL_PAD,), _f32)
    z128 = jnp.zeros((NPAD, 128), _f32)
    zpool = jnp.zeros((POOL_PAD, 128), _f32)

    degp, cntp = sc_hist(dstE, batchE, ones_hbm, z1, zc)

    E_BLK = 320
    idxt = pl.pallas_call(
        _idxt_body,
        grid=(EROWS // E_BLK,),
        in_specs=[pl.BlockSpec((E_BLK, 128), lambda i: (i, 0))],
        out_specs=pl.BlockSpec((NQ, E_BLK, 128), lambda i: (0, i, 0)),
        out_shape=jax.ShapeDtypeStruct((NQ, EROWS, 128), jnp.int32),
    )(dstE)
    idxtE = idxt.reshape(NQ * EROWS, 128)

    dinv2, t02 = pl.pallas_call(
        _prep_body,
        out_shape=[jax.ShapeDtypeStruct((NROWS, 128), _f32)] * 2,
    )(degp.reshape(NC, NROWS, 128), x_flat.reshape(NROWS, 128),
      mask_flat.reshape(NROWS, 128))
    dinvc = dinv2.reshape(NPAD, 1)
    maskc = mask_flat.reshape(NPAD, 1)
    t0_flat = t02.reshape(NPAD)

    s0p = sc_s0(srcE, dstE, t0_flat, z1)

    BLK = 512
    grid = (NPAD // BLK,)
    W1p = jnp.zeros((1, 128), _f32).at[0, :64].set(W1[0])
    b1p = jnp.zeros((1, 128), _f32).at[0, :64].set(b1)
    W2p = jnp.zeros((128, 128), _f32).at[:64].set(W2)
    t1 = pl.pallas_call(
        _l1_body,
        grid=grid,
        in_specs=[
            pl.BlockSpec((NC, BLK, 1), lambda i: (0, i, 0)),
            _node_spec(BLK, 1), _node_spec(BLK, 1), _node_spec(BLK, 1),
            pl.BlockSpec((1, 128), lambda i: (0, 0)),
            pl.BlockSpec((1, 128), lambda i: (0, 0)),
        ],
        out_specs=_node_spec(BLK, 128),
        out_shape=jax.ShapeDtypeStruct((NPAD, 128), _f32),
    )(s0p.reshape(NC, NPAD, 1), t0_flat.reshape(NPAD, 1), dinvc, maskc,
      W1p, b1p)

    s1 = sc_agg(srcE, idxtE, t1, z128)

    t2 = pl.pallas_call(
        _l2_body,
        grid=grid,
        in_specs=[
            _node_spec(BLK, 128), _node_spec(BLK, 128),
            _node_spec(BLK, 1), _node_spec(BLK, 1),
            pl.BlockSpec((128, 128), lambda i: (0, 0)),
            pl.BlockSpec((1, 128), lambda i: (0, 0)),
        ],
        out_specs=_node_spec(BLK, 128),
        out_shape=jax.ShapeDtypeStruct((NPAD, 128), _f32),
    )(s1, t1, dinvc, maskc, W2p, b2.reshape(1, 128))

    s2 = sc_agg(srcE, idxtE, t2, z128)

    a2 = pl.pallas_call(
        _l3_body,
        grid=grid,
        in_specs=[_node_spec(BLK, 128)] * 2 + [_node_spec(BLK, 1)],
        out_specs=_node_spec(BLK, 128),
        out_shape=jax.ShapeDtypeStruct((NPAD, 128), _f32),
    )(s2, t2, dinvc)

    poolp = sc_pool(a2, batchE, zpool)

    f2Wp = jnp.zeros((128, 128), _f32).at[:, :12].set(fc2_W)
    f2bp = jnp.zeros((1, 128), _f32).at[0, :12].set(fc2_b)
    out_p = pl.pallas_call(
        _head_body,
        out_shape=jax.ShapeDtypeStruct((NG, 128), _f32),
    )(poolp[:, :NG, :], cntp[:, :NG].reshape(NC, NG, 1), W3,
      b3.reshape(1, 256), fc1_W, fc1_b.reshape(1, 128), f2Wp, f2bp)
    return out_p[:, :12]


# speedup vs baseline: 13.0080x; 2.2050x over previous
import functools

import jax
import jax.numpy as jnp
from jax import lax
from jax.experimental import pallas as pl
from jax.experimental.pallas import tpu as pltpu
from jax.experimental.pallas import tpu_sc as plsc

N = 50000
NE = 800000
NG = 1024

NPAD = 53248
NROWS = NPAD // 128
NE_P = 819200
EROWS = NE_P // 128
POOL_PAD = 1152
NC, NS = 2, 16
ROWS_T = NPAD // NS
CROWS = 72
NQ1 = 4
QR1 = 12512
HQ = QR1 // 2
ACC1 = 6272
WR1 = 392
S1ROWS = NPAD // 2

_f32 = jnp.float32


def _ds(ref, start, size):
    return ref.at[pl.ds(start, size)]


@functools.lru_cache(maxsize=None)
def _build_sc_kernels():
    mesh = plsc.VectorSubcoreMesh(core_axis_name="c", subcore_axis_name="s")

    G_H = 8
    @functools.partial(
        pl.kernel,
        mesh=mesh,
        out_type=[
            jax.ShapeDtypeStruct((NC, NPAD), _f32),
            jax.ShapeDtypeStruct((NC, POOL_PAD), _f32),
        ],
        scratch_types=[
            pltpu.VMEM_SHARED((NPAD,), _f32),
            pltpu.VMEM_SHARED((POOL_PAD,), _f32),
            pltpu.VMEM((16, 128), _f32),
            pltpu.VMEM((G_H, 128), jnp.int32),
            pltpu.VMEM((G_H, 128), jnp.int32),
            pltpu.VMEM((16, 128), jnp.int32),
            pltpu.SemaphoreType.DMA,
        ],
    )
    def sc_hist(dstE, batchE, ones_hbm, z1, zc, degp, cntp,
                acc_deg, acc_cnt, ones_v, idx0, idx1, idxb_v, sem_s):
        c = lax.axis_index("c")
        s = lax.axis_index("s")
        w = s * NC + c
        idxs = [idx0, idx1]
        pltpu.sync_copy(_ds(z1, s * ROWS_T, ROWS_T), _ds(acc_deg, s * ROWS_T, ROWS_T))

        @pl.when(s < POOL_PAD // 128)
        def _():
            pltpu.sync_copy(_ds(zc, s * 128, 128), _ds(acc_cnt, s * 128, 128))
        pltpu.sync_copy(ones_hbm, ones_v)
        plsc.subcore_barrier()

        pltpu.sync_copy(_ds(dstE, w * 200, G_H), idx0)

        def scat_group(buf):
            for j in range(G_H):
                pltpu.async_copy(ones_v.at[j], acc_deg.at[buf.at[j]],
                                 sem_s, add=True)
            for j in range(G_H):
                pltpu.make_async_copy(z1.at[pl.ds(0, 128)],
                                      ones_v.at[j], sem_s).wait()

        def outer(o, carry):
            for b in (0, 1):
                g = o * 2 + b

                @pl.when(g < 24)
                def _():
                    pltpu.sync_copy(_ds(dstE, w * 200 + (g + 1) * G_H, G_H),
                                    idxs[1 - b])
                scat_group(idxs[b])
            return carry

        lax.fori_loop(0, 12, outer, 0)
        scat_group(idx0)

        @pl.when(w < 26)
        def _():
            pltpu.sync_copy(_ds(batchE, w * 16, 16), idxb_v)
            for j in range(16):
                pltpu.sync_copy(ones_v.at[j], acc_cnt.at[idxb_v.at[j]], add=True)
        plsc.subcore_barrier()
        pltpu.sync_copy(_ds(acc_deg, s * ROWS_T, ROWS_T),
                        degp.at[c].at[pl.ds(s * ROWS_T, ROWS_T)])

        @pl.when(s < POOL_PAD // 128)
        def _():
            pltpu.sync_copy(_ds(acc_cnt, s * 128, 128),
                            cntp.at[c].at[pl.ds(s * 128, 128)])

    G_0 = 8
    @functools.partial(
        pl.kernel,
        mesh=mesh,
        out_type=jax.ShapeDtypeStruct((NC, NPAD), _f32),
        scratch_types=[
            pltpu.VMEM_SHARED((NPAD,), _f32),
            pltpu.VMEM((G_0, 128), jnp.int32),
            pltpu.VMEM((G_0, 128), jnp.int32),
            pltpu.VMEM((G_0, 128), jnp.int32),
            pltpu.VMEM((G_0, 128), jnp.int32),
            pltpu.VMEM((G_0, 128), _f32),
            pltpu.VMEM((G_0, 128), _f32),
            pltpu.SemaphoreType.DMA,
            pltpu.SemaphoreType.DMA,
        ],
    )
    def sc_s0(srcE, dstE, t0, z1, s0p, acc,
              idxs0, idxs1, idxd0, idxd1, vals0, vals1, sem_g, sem_s):
        c = lax.axis_index("c")
        s = lax.axis_index("s")
        w = s * NC + c
        idxs = [idxs0, idxs1]
        idxd = [idxd0, idxd1]
        vals = [vals0, vals1]
        pltpu.sync_copy(_ds(z1, s * ROWS_T, ROWS_T), _ds(acc, s * ROWS_T, ROWS_T))
        plsc.subcore_barrier()

        def load_idx(g, b):
            off = w * 200 + g * G_0
            pltpu.sync_copy(_ds(srcE, off, G_0), idxs[b])
            pltpu.sync_copy(_ds(dstE, off, G_0), idxd[b])

        def fire_gathers(b):
            for j in range(G_0):
                pltpu.async_copy(t0.at[idxs[b].at[j]], vals[b].at[j], sem_g)

        load_idx(0, 0)
        fire_gathers(0)

        def drain_group(b):
            for j in range(G_0):
                pltpu.make_async_copy(z1.at[pl.ds(0, 128)],
                                      vals[b].at[j], sem_g).wait()

        def scat_group(b):
            for j in range(G_0):
                pltpu.async_copy(vals[b].at[j], acc.at[idxd[b].at[j]],
                                 sem_s, add=True)
            for j in range(G_0):
                pltpu.make_async_copy(z1.at[pl.ds(0, 128)],
                                      vals[b].at[j], sem_s).wait()

        def outer(o, carry):
            for b in (0, 1):
                g = o * 2 + b
                drain_group(b)

                @pl.when(g < 24)
                def _():
                    load_idx(g + 1, 1 - b)
                    fire_gathers(1 - b)
                scat_group(b)
            return carry

        lax.fori_loop(0, 12, outer, 0)
        drain_group(0)
        scat_group(0)
        plsc.subcore_barrier()
        pltpu.sync_copy(_ds(acc, s * ROWS_T, ROWS_T),
                        s0p.at[c].at[pl.ds(s * ROWS_T, ROWS_T)])

    @functools.partial(
        pl.kernel,
        mesh=mesh,
        out_type=jax.ShapeDtypeStruct((S1ROWS, 128), _f32),
        scratch_types=[
            pltpu.VMEM_SHARED((ACC1, 128), _f32),
            pltpu.VMEM((8, 128), jnp.int32),
            pltpu.VMEM((8, 128), jnp.int32),
            pltpu.VMEM((128, 128), _f32),
            pltpu.VMEM((128, 128), _f32),
            pltpu.SemaphoreType.DMA,
            pltpu.SemaphoreType.DMA,
        ],
    )
    def sc_agg1(gsrcE, idxpE, tab, z128, out,
                acc, idxs, idxd, vals0, vals1, sem_g, sem_s):
        c = lax.axis_index("c")
        s = lax.axis_index("s")
        vals = [vals0, vals1]

        for p in range(NQ1 // NC):
            pltpu.sync_copy(_ds(z128, s * (ACC1 // NS), ACC1 // NS),
                            _ds(acc, s * (ACC1 // NS), ACC1 // NS))
            plsc.subcore_barrier()

            for ci in range(NC):
                @pl.when(c == ci)
                def _(_ci=ci, _p=p):
                    q = _ci * (NQ1 // NC) + _p

                    def body(g, carry):
                        off = s * 400 + g * 8
                        pltpu.sync_copy(_ds(gsrcE, off, 8), idxs)
                        pltpu.sync_copy(
                            _ds(idxpE, q * EROWS + off, 8), idxd)
                        pltpu.async_copy(tab.at[idxs.at[0]], vals0, sem_g)
                        for j in range(8):
                            b = j & 1
                            pltpu.make_async_copy(tab.at[pl.ds(0, 128)],
                                                  vals[b], sem_g).wait()
                            if j < 7:
                                if j >= 1:
                                    pltpu.make_async_copy(
                                        tab.at[pl.ds(0, 128)],
                                        vals[1 - b], sem_s).wait()
                                pltpu.async_copy(tab.at[idxs.at[j + 1]],
                                                 vals[1 - b], sem_g)
                            pltpu.async_copy(vals[b], acc.at[idxd.at[j]],
                                             sem_s, add=True)
                        for b in (0, 1):
                            pltpu.make_async_copy(tab.at[pl.ds(0, 128)],
                                                  vals[b], sem_s).wait()
                        return carry

                    lax.fori_loop(0, 50, body, 0)
            plsc.subcore_barrier()
            for ci in range(NC):
                @pl.when(c == ci)
                def _(_ci=ci, _p=p):
                    q = _ci * (NQ1 // NC) + _p

                    @pl.when(s < NS - 1)
                    def _():
                        pltpu.sync_copy(
                            _ds(acc, s * WR1, WR1),
                            out.at[pl.ds(q * HQ + s * WR1, WR1)])

                    @pl.when(s == NS - 1)
                    def _():
                        pltpu.sync_copy(
                            _ds(acc, (NS - 1) * WR1, HQ - (NS - 1) * WR1),
                            out.at[pl.ds(q * HQ + (NS - 1) * WR1,
                                         HQ - (NS - 1) * WR1)])
            plsc.subcore_barrier()

    @functools.partial(
        pl.kernel,
        mesh=mesh,
        out_type=jax.ShapeDtypeStruct((NC, POOL_PAD, 128), _f32),
        scratch_types=[
            pltpu.VMEM_SHARED((POOL_PAD, 128), _f32),
            pltpu.VMEM((8, 128), jnp.int32),
            pltpu.VMEM((8, 128), jnp.int32),
            pltpu.VMEM((8, 128), _f32),
            pltpu.VMEM((8, 128), jnp.int32),
            pltpu.VMEM((128, 128), _f32),
            pltpu.VMEM((128, 128), _f32),
            pltpu.SemaphoreType.DMA,
            pltpu.SemaphoreType.DMA,
        ],
    )
    def sc_agg2g(srcE, dstE, t2h, dinvF, batchF, zpool, outp,
                 acc, idxs, idxd, dv, gi, vals0, vals1, sem_g, sem_s):
        c = lax.axis_index("c")
        s = lax.axis_index("s")
        w = s * NC + c
        vals = [vals0, vals1]
        pltpu.sync_copy(_ds(zpool, s * CROWS, CROWS), _ds(acc, s * CROWS, CROWS))
        plsc.subcore_barrier()

        def group(g, carry):
            off = w * 200 + g * 8
            pltpu.sync_copy(_ds(srcE, off, 8), idxs)
            pltpu.sync_copy(_ds(dstE, off, 8), idxd)
            for j in range(8):
                pltpu.async_copy(dinvF.at[idxd.at[j]], dv.at[j], sem_g)
                pltpu.async_copy(batchF.at[idxd.at[j]], gi.at[j], sem_g)
            for j in range(8):
                pltpu.make_async_copy(dinvF.at[pl.ds(0, 128)],
                                      dv.at[j], sem_g).wait()
                pltpu.make_async_copy(batchF.at[pl.ds(0, 128)],
                                      gi.at[j], sem_g).wait()
            pltpu.async_copy(t2h.at[idxs.at[0]], vals0, sem_g)
            for j in range(8):
                b = j & 1
                pltpu.make_async_copy(t2h.at[pl.ds(0, 128)],
                                      vals[b], sem_g).wait()
                if j < 7:
                    if j >= 1:
                        pltpu.make_async_copy(t2h.at[pl.ds(0, 128)],
                                              vals[1 - b], sem_s).wait()
                    pltpu.async_copy(t2h.at[idxs.at[j + 1]],
                                     vals[1 - b], sem_g)

                def rbody(rb, cc, _vb=vals[b], _j=j):
                    base = pl.multiple_of(rb * 16, 16)
                    mv = dv[_j, pl.ds(base, 16)]
                    for i in range(16):
                        m = mv[i]
                        r = base + i
                        for k in range(8):
                            sl = pl.ds(k * 16, 16)
                            _vb[r, sl] = _vb[r, sl] * m
                    return cc

                lax.fori_loop(0, 8, rbody, 0)
                pltpu.async_copy(vals[b], acc.at[gi.at[j]], sem_s, add=True)
            for b in (0, 1):
                pltpu.make_async_copy(t2h.at[pl.ds(0, 128)],
                                      vals[b], sem_s).wait()
            return carry

        lax.fori_loop(0, 25, group, 0)
        plsc.subcore_barrier()
        pltpu.sync_copy(_ds(acc, s * CROWS, CROWS),
                        outp.at[c].at[pl.ds(s * CROWS, CROWS)])

    @functools.partial(
        pl.kernel,
        mesh=mesh,
        out_type=jax.ShapeDtypeStruct((NC, POOL_PAD, 128), _f32),
        scratch_types=[
            pltpu.VMEM_SHARED((POOL_PAD, 128), _f32),
            pltpu.VMEM((16, 128), jnp.int32),
            pltpu.VMEM((128, 128), _f32),
            pltpu.VMEM((128, 128), _f32),
            pltpu.SemaphoreType.DMA,
        ],
    )
    def sc_pool(a2, batchE, zpool, poolp, acc, idx_v, vals0, vals1, sem_s):
        c = lax.axis_index("c")
        s = lax.axis_index("s")
        w = s * NC + c
        vals = [vals0, vals1]
        pltpu.sync_copy(_ds(zpool, s * CROWS, CROWS), _ds(acc, s * CROWS, CROWS))
        plsc.subcore_barrier()

        @pl.when(w < 26)
        def _():
            pltpu.sync_copy(_ds(batchE, w * 16, 16), idx_v)
            for k in range(16):
                b = k & 1
                if k >= 2:
                    pltpu.make_async_copy(_ds(a2, 0, 128), vals[b], sem_s).wait()
                pltpu.sync_copy(_ds(a2, w * 2048 + k * 128, 128), vals[b])
                pltpu.async_copy(vals[b], acc.at[idx_v.at[k]], sem_s, add=True)
            for k in (14, 15):
                pltpu.make_async_copy(_ds(a2, 0, 128), vals[k & 1], sem_s).wait()
        plsc.subcore_barrier()
        pltpu.sync_copy(_ds(acc, s * CROWS, CROWS),
                        poolp.at[c].at[pl.ds(s * CROWS, CROWS)])

    return sc_hist, sc_s0, sc_agg1, sc_agg2g, sc_pool


def _prep_body(degp_ref, x_ref, mask_ref, dinv_ref, t0_ref):
    deg = degp_ref[0] + degp_ref[1] + 1.0
    dinv = lax.rsqrt(deg)
    dinv_ref[...] = dinv
    t0_ref[...] = dinv * x_ref[...] * mask_ref[...]


def _l1_body(s0p_ref, t0_ref, dinv_ref, mask_ref, W1_ref, b1_ref,
             t1a_ref, t1b_ref, t1c_ref):
    s0 = s0p_ref[0] + s0p_ref[1]
    dinv = dinv_ref[...]
    a0 = dinv * (s0 + t0_ref[...])
    h1 = jnp.maximum(a0 * W1_ref[...] + b1_ref[...], 0.0)
    t1 = jnp.where(mask_ref[...] > 0.0, dinv * h1, 0.0)
    t1a_ref[...] = t1
    t1b_ref[...] = jnp.concatenate([t1[:, 64:], t1[:, :64]], axis=1)
    t1c_ref[...] = t1[:, :64]


def _l2_body(s1_ref, t1c_ref, dinv_ref, mask_ref, W2_ref, b2_ref,
             t2_ref, u2_ref):
    dinv = dinv_ref[...]
    a1 = dinv * (s1_ref[...] + t1c_ref[...])
    h2 = jnp.dot(a1, W2_ref[...], preferred_element_type=_f32)
    h2 = jnp.maximum(h2 + b2_ref[...], 0.0)
    t2 = jnp.where(mask_ref[...] > 0.0, dinv * h2, 0.0)
    t2_ref[...] = t2
    u2_ref[...] = dinv * t2


def _idx_body(srcE_ref, dstE_ref, gsrc_ref, idxp_ref):
    d = dstE_ref[...]
    gsrc_ref[...] = srcE_ref[...] + NPAD * (d & 1)
    for q in range(NQ1):
        lo = q * QR1
        loc = jnp.where((d >= lo) & (d < lo + QR1), (d - lo) >> 1, HQ)
        idxp_ref[q, :, :] = loc


def _head_body(poolp_ref, e2p_ref, cntp_ref, W3_ref, b3_ref, f1W_ref, f1b_ref,
               f2W_ref, f2b_ref, out_ref):
    pool = (poolp_ref[0] + poolp_ref[1]) + (e2p_ref[0] + e2p_ref[1])
    cnt = cntp_ref[0] + cntp_ref[1]
    g = pool / jnp.maximum(cnt, 1.0)
    G = jnp.dot(g, W3_ref[...], preferred_element_type=_f32) + b3_ref[...]
    G = jnp.maximum(jnp.dot(G, f1W_ref[...], preferred_element_type=_f32)
                    + f1b_ref[...], 0.0)
    out_ref[...] = jnp.dot(G, f2W_ref[...], preferred_element_type=_f32) + f2b_ref[...]


def _node_spec(blk, ncols):
    return pl.BlockSpec((blk, ncols), lambda i: (i, 0))


def kernel(x, edge_index, batch, W1, b1, W2, b2, W3, b3, fc1_W, fc1_b, fc2_W, fc2_b):
    sc_hist, sc_s0, sc_agg1, sc_agg2g, sc_pool = _build_sc_kernels()
    src = edge_index[0].astype(jnp.int32)
    dst = edge_index[1].astype(jnp.int32)
    batch = batch.astype(jnp.int32)

    epad = 50000 + (jnp.arange(NE_P - NE, dtype=jnp.int32) % (NPAD - N))
    srcE = jnp.concatenate([src, epad]).reshape(EROWS, 128)
    dstE = jnp.concatenate([dst, epad]).reshape(EROWS, 128)
    bpad = NG + (jnp.arange(NPAD - N, dtype=jnp.int32) % 32)
    batch_flat = jnp.concatenate([batch, bpad])
    batchE = batch_flat.reshape(NROWS, 128)
    x_flat = jnp.concatenate([x[:, 0], jnp.zeros((NPAD - N,), _f32)])
    mask_flat = (jnp.arange(NPAD) < N).astype(_f32)

    ones_hbm = jnp.ones((16, 128), _f32)
    z1 = jnp.zeros((NPAD,), _f32)
    zc = jnp.zeros((POOL_PAD,), _f32)
    z128 = jnp.zeros((NPAD, 128), _f32)
    zpool = jnp.zeros((POOL_PAD, 128), _f32)

    degp, cntp = sc_hist(dstE, batchE, ones_hbm, z1, zc)

    E_BLK = 320
    gsrc, idxp = pl.pallas_call(
        _idx_body,
        grid=(EROWS // E_BLK,),
        in_specs=[pl.BlockSpec((E_BLK, 128), lambda i: (i, 0))] * 2,
        out_specs=[pl.BlockSpec((E_BLK, 128), lambda i: (i, 0)),
                   pl.BlockSpec((NQ1, E_BLK, 128), lambda i: (0, i, 0))],
        out_shape=[jax.ShapeDtypeStruct((EROWS, 128), jnp.int32),
                   jax.ShapeDtypeStruct((NQ1, EROWS, 128), jnp.int32)],
    )(srcE, dstE)
    idxpE = idxp.reshape(NQ1 * EROWS, 128)

    dinv2, t02 = pl.pallas_call(
        _prep_body,
        out_shape=[jax.ShapeDtypeStruct((NROWS, 128), _f32)] * 2,
    )(degp.reshape(NC, NROWS, 128), x_flat.reshape(NROWS, 128),
      mask_flat.reshape(NROWS, 128))
    dinvc = dinv2.reshape(NPAD, 1)
    dinv_flat = dinv2.reshape(NPAD)
    maskc = mask_flat.reshape(NPAD, 1)
    t0_flat = t02.reshape(NPAD)

    s0p = sc_s0(srcE, dstE, t0_flat, z1)

    BLK = 512
    grid = (NPAD // BLK,)
    W1p = jnp.zeros((1, 128), _f32).at[0, :64].set(W1[0])
    b1p = jnp.zeros((1, 128), _f32).at[0, :64].set(b1)
    t1a, t1b, t1c = pl.pallas_call(
        _l1_body,
        grid=grid,
        in_specs=[
            pl.BlockSpec((NC, BLK, 1), lambda i: (0, i, 0)),
            _node_spec(BLK, 1), _node_spec(BLK, 1), _node_spec(BLK, 1),
            pl.BlockSpec((1, 128), lambda i: (0, 0)),
            pl.BlockSpec((1, 128), lambda i: (0, 0)),
        ],
        out_specs=[_node_spec(BLK, 128), _node_spec(BLK, 128),
                   _node_spec(BLK, 64)],
        out_shape=[jax.ShapeDtypeStruct((NPAD, 128), _f32),
                   jax.ShapeDtypeStruct((NPAD, 128), _f32),
                   jax.ShapeDtypeStruct((NPAD, 64), _f32)],
    )(s0p.reshape(NC, NPAD, 1), t0_flat.reshape(NPAD, 1), dinvc, maskc,
      W1p, b1p)
    Q = jnp.concatenate([t1a, t1b], axis=0)

    s1p = sc_agg1(gsrc, idxpE, Q, z128)
    s1_64 = s1p.reshape(NPAD, 64)

    t2, u2 = pl.pallas_call(
        _l2_body,
        grid=grid,
        in_specs=[
            _node_spec(BLK, 64), _node_spec(BLK, 64),
            _node_spec(BLK, 1), _node_spec(BLK, 1),
            pl.BlockSpec((64, 128), lambda i: (0, 0)),
            pl.BlockSpec((1, 128), lambda i: (0, 0)),
        ],
        out_specs=[_node_spec(BLK, 128)] * 2,
        out_shape=[jax.ShapeDtypeStruct((NPAD, 128), _f32)] * 2,
    )(s1_64, t1c, dinvc, maskc, W2, b2.reshape(1, 128))

    e2p = sc_agg2g(srcE, dstE, t2, dinv_flat, batch_flat, zpool)

    poolp = sc_pool(u2, batchE, zpool)

    f2Wp = jnp.zeros((128, 128), _f32).at[:, :12].set(fc2_W)
    f2bp = jnp.zeros((1, 128), _f32).at[0, :12].set(fc2_b)
    out_p = pl.pallas_call(
        _head_body,
        out_shape=jax.ShapeDtypeStruct((NG, 128), _f32),
    )(poolp[:, :NG, :], e2p[:, :NG, :], cntp[:, :NG].reshape(NC, NG, 1), W3,
      b3.reshape(1, 256), fc1_W, fc1_b.reshape(1, 128), f2Wp, f2bp)
    return out_p[:, :12]


# speedup vs baseline: 18.3751x; 1.4126x over previous
import functools

import jax
import jax.numpy as jnp
from jax import lax
from jax.experimental import pallas as pl
from jax.experimental.pallas import tpu as pltpu
from jax.experimental.pallas import tpu_sc as plsc

N = 50000
NE = 800000
NG = 1024

NPAD = 53248
NROWS = NPAD // 128
NE_P = 819200
EROWS = NE_P // 128
POOL_PAD = 1152
NC, NS = 2, 16
ROWS_T = NPAD // NS
CROWS = 72
NQ1 = 2
QR1 = 25024
HQ = QR1 // 2
ACC1 = 12544
WR1 = 784
S1ROWS = NPAD // 2

_f32 = jnp.float32


def _ds(ref, start, size):
    return ref.at[pl.ds(start, size)]


@functools.lru_cache(maxsize=None)
def _build_sc_kernels():
    mesh = plsc.VectorSubcoreMesh(core_axis_name="c", subcore_axis_name="s")

    G_H = 8
    @functools.partial(
        pl.kernel,
        mesh=mesh,
        out_type=[
            jax.ShapeDtypeStruct((NC, NPAD), _f32),
            jax.ShapeDtypeStruct((NC, POOL_PAD), _f32),
        ],
        scratch_types=[
            pltpu.VMEM_SHARED((NPAD,), _f32),
            pltpu.VMEM_SHARED((POOL_PAD,), _f32),
            pltpu.VMEM((16, 128), _f32),
            pltpu.VMEM((G_H, 128), jnp.int32),
            pltpu.VMEM((G_H, 128), jnp.int32),
            pltpu.VMEM((16, 128), jnp.int32),
            pltpu.SemaphoreType.DMA,
        ],
    )
    def sc_hist(dstE, batchE, ones_hbm, z1, zc, degp, cntp,
                acc_deg, acc_cnt, ones_v, idx0, idx1, idxb_v, sem_s):
        c = lax.axis_index("c")
        s = lax.axis_index("s")
        w = s * NC + c
        idxs = [idx0, idx1]
        pltpu.sync_copy(_ds(z1, s * ROWS_T, ROWS_T), _ds(acc_deg, s * ROWS_T, ROWS_T))

        @pl.when(s < POOL_PAD // 128)
        def _():
            pltpu.sync_copy(_ds(zc, s * 128, 128), _ds(acc_cnt, s * 128, 128))
        pltpu.sync_copy(ones_hbm, ones_v)
        plsc.subcore_barrier()

        pltpu.sync_copy(_ds(dstE, w * 200, G_H), idx0)

        def scat_group(buf):
            for j in range(G_H):
                pltpu.async_copy(ones_v.at[j], acc_deg.at[buf.at[j]],
                                 sem_s, add=True)
            for j in range(G_H):
                pltpu.make_async_copy(z1.at[pl.ds(0, 128)],
                                      ones_v.at[j], sem_s).wait()

        def outer(o, carry):
            for b in (0, 1):
                g = o * 2 + b

                @pl.when(g < 24)
                def _():
                    pltpu.sync_copy(_ds(dstE, w * 200 + (g + 1) * G_H, G_H),
                                    idxs[1 - b])
                scat_group(idxs[b])
            return carry

        lax.fori_loop(0, 12, outer, 0)
        scat_group(idx0)

        @pl.when(w < 26)
        def _():
            pltpu.sync_copy(_ds(batchE, w * 16, 16), idxb_v)
            for j in range(16):
                pltpu.sync_copy(ones_v.at[j], acc_cnt.at[idxb_v.at[j]], add=True)
        plsc.subcore_barrier()
        pltpu.sync_copy(_ds(acc_deg, s * ROWS_T, ROWS_T),
                        degp.at[c].at[pl.ds(s * ROWS_T, ROWS_T)])

        @pl.when(s < POOL_PAD // 128)
        def _():
            pltpu.sync_copy(_ds(acc_cnt, s * 128, 128),
                            cntp.at[c].at[pl.ds(s * 128, 128)])

    G_0 = 8
    @functools.partial(
        pl.kernel,
        mesh=mesh,
        out_type=jax.ShapeDtypeStruct((NC, NPAD), _f32),
        scratch_types=[
            pltpu.VMEM_SHARED((NPAD,), _f32),
            pltpu.VMEM((G_0, 128), jnp.int32),
            pltpu.VMEM((G_0, 128), jnp.int32),
            pltpu.VMEM((G_0, 128), jnp.int32),
            pltpu.VMEM((G_0, 128), jnp.int32),
            pltpu.VMEM((G_0, 128), _f32),
            pltpu.VMEM((G_0, 128), _f32),
            pltpu.SemaphoreType.DMA,
            pltpu.SemaphoreType.DMA,
        ],
    )
    def sc_s0(srcE, dstE, t0, z1, s0p, acc,
              idxs0, idxs1, idxd0, idxd1, vals0, vals1, sem_g, sem_s):
        c = lax.axis_index("c")
        s = lax.axis_index("s")
        w = s * NC + c
        idxs = [idxs0, idxs1]
        idxd = [idxd0, idxd1]
        vals = [vals0, vals1]
        pltpu.sync_copy(_ds(z1, s * ROWS_T, ROWS_T), _ds(acc, s * ROWS_T, ROWS_T))
        plsc.subcore_barrier()

        def load_idx(g, b):
            off = w * 200 + g * G_0
            pltpu.sync_copy(_ds(srcE, off, G_0), idxs[b])
            pltpu.sync_copy(_ds(dstE, off, G_0), idxd[b])

        def fire_gathers(b):
            for j in range(G_0):
                pltpu.async_copy(t0.at[idxs[b].at[j]], vals[b].at[j], sem_g)

        load_idx(0, 0)
        fire_gathers(0)

        def drain_group(b):
            for j in range(G_0):
                pltpu.make_async_copy(z1.at[pl.ds(0, 128)],
                                      vals[b].at[j], sem_g).wait()

        def scat_group(b):
            for j in range(G_0):
                pltpu.async_copy(vals[b].at[j], acc.at[idxd[b].at[j]],
                                 sem_s, add=True)
            for j in range(G_0):
                pltpu.make_async_copy(z1.at[pl.ds(0, 128)],
                                      vals[b].at[j], sem_s).wait()

        def outer(o, carry):
            for b in (0, 1):
                g = o * 2 + b
                drain_group(b)

                @pl.when(g < 24)
                def _():
                    load_idx(g + 1, 1 - b)
                    fire_gathers(1 - b)
                scat_group(b)
            return carry

        lax.fori_loop(0, 12, outer, 0)
        drain_group(0)
        scat_group(0)
        plsc.subcore_barrier()
        pltpu.sync_copy(_ds(acc, s * ROWS_T, ROWS_T),
                        s0p.at[c].at[pl.ds(s * ROWS_T, ROWS_T)])

    @functools.partial(
        pl.kernel,
        mesh=mesh,
        out_type=jax.ShapeDtypeStruct((S1ROWS, 128), _f32),
        scratch_types=[
            pltpu.VMEM_SHARED((ACC1, 128), _f32),
            pltpu.VMEM((8, 128), jnp.int32),
            pltpu.VMEM((8, 128), jnp.int32),
            pltpu.VMEM((64, 128), _f32),
            pltpu.VMEM((64, 128), _f32),
            pltpu.SemaphoreType.DMA,
            pltpu.SemaphoreType.DMA,
        ],
    )
    def sc_agg1(gsrcE, idxpE, tab, z128, out,
                acc, idxs, idxd, vals0, vals1, sem_g, sem_s):
        c = lax.axis_index("c")
        s = lax.axis_index("s")
        vals = [vals0, vals1]

        pltpu.sync_copy(_ds(z128, s * (ACC1 // NS), ACC1 // NS),
                        _ds(acc, s * (ACC1 // NS), ACC1 // NS))
        plsc.subcore_barrier()

        def ihalf(ref, h):
            return ref.at[h >> 1].at[pl.ds((h & 1) * 64, 64)]

        for ci in range(NC):
            @pl.when(c == ci)
            def _(_ci=ci):
                def body(g, carry):
                    off = s * 400 + g * 8
                    pltpu.sync_copy(_ds(gsrcE, off, 8), idxs)
                    pltpu.sync_copy(
                        _ds(idxpE, _ci * EROWS + off, 8), idxd)
                    pltpu.async_copy(tab.at[ihalf(idxs, 0)], vals0, sem_g)
                    for h in range(16):
                        b = h & 1
                        pltpu.make_async_copy(tab.at[pl.ds(0, 64)],
                                              vals[b], sem_g).wait()
                        if h < 15:
                            if h >= 1:
                                pltpu.make_async_copy(
                                    tab.at[pl.ds(0, 64)],
                                    vals[1 - b], sem_s).wait()
                            pltpu.async_copy(tab.at[ihalf(idxs, h + 1)],
                                             vals[1 - b], sem_g)
                        pltpu.async_copy(vals[b], acc.at[ihalf(idxd, h)],
                                         sem_s, add=True)
                    for b in (0, 1):
                        pltpu.make_async_copy(tab.at[pl.ds(0, 64)],
                                              vals[b], sem_s).wait()
                    return carry

                lax.fori_loop(0, 50, body, 0)
        plsc.subcore_barrier()

        for ci in range(NC):
            @pl.when(c == ci)
            def _(_ci=ci):
                @pl.when(s < NS - 1)
                def _():
                    pltpu.sync_copy(_ds(acc, s * WR1, WR1),
                                    out.at[pl.ds(_ci * HQ + s * WR1, WR1)])

                @pl.when(s == NS - 1)
                def _():
                    pltpu.sync_copy(
                        _ds(acc, (NS - 1) * WR1, HQ - (NS - 1) * WR1),
                        out.at[pl.ds(_ci * HQ + (NS - 1) * WR1,
                                     HQ - (NS - 1) * WR1)])

    @functools.partial(
        pl.kernel,
        mesh=mesh,
        out_type=jax.ShapeDtypeStruct((NC, POOL_PAD, 128), _f32),
        scratch_types=[
            pltpu.VMEM_SHARED((POOL_PAD, 128), _f32),
            pltpu.VMEM((8, 128), jnp.int32),
            pltpu.VMEM((8, 128), jnp.int32),
            pltpu.VMEM((8, 128), _f32),
            pltpu.VMEM((8, 128), jnp.int32),
            pltpu.VMEM((128, 128), _f32),
            pltpu.VMEM((128, 128), _f32),
            pltpu.SemaphoreType.DMA,
            pltpu.SemaphoreType.DMA,
        ],
    )
    def sc_agg2g(srcE, dstE, t2h, dinvF, batchF, zpool, outp,
                 acc, idxs, idxd, dv, gi, vals0, vals1, sem_g, sem_s):
        c = lax.axis_index("c")
        s = lax.axis_index("s")
        w = s * NC + c
        vals = [vals0, vals1]
        pltpu.sync_copy(_ds(zpool, s * CROWS, CROWS), _ds(acc, s * CROWS, CROWS))
        plsc.subcore_barrier()

        def group(g, carry):
            off = w * 200 + g * 8
            pltpu.sync_copy(_ds(srcE, off, 8), idxs)
            pltpu.sync_copy(_ds(dstE, off, 8), idxd)
            for j in range(8):
                pltpu.async_copy(dinvF.at[idxd.at[j]], dv.at[j], sem_g)
                pltpu.async_copy(batchF.at[idxd.at[j]], gi.at[j], sem_g)
            for j in range(8):
                pltpu.make_async_copy(dinvF.at[pl.ds(0, 128)],
                                      dv.at[j], sem_g).wait()
                pltpu.make_async_copy(batchF.at[pl.ds(0, 128)],
                                      gi.at[j], sem_g).wait()
            pltpu.async_copy(t2h.at[idxs.at[0]], vals0, sem_g)
            for j in range(8):
                b = j & 1
                pltpu.make_async_copy(t2h.at[pl.ds(0, 128)],
                                      vals[b], sem_g).wait()
                if j < 7:
                    if j >= 1:
                        pltpu.make_async_copy(t2h.at[pl.ds(0, 128)],
                                              vals[1 - b], sem_s).wait()
                    pltpu.async_copy(t2h.at[idxs.at[j + 1]],
                                     vals[1 - b], sem_g)

                def rbody(rb, cc, _vb=vals[b], _j=j):
                    base = pl.multiple_of(rb * 16, 16)
                    mv = dv[_j, pl.ds(base, 16)]
                    for i in range(16):
                        m = mv[i]
                        r = base + i
                        for k in range(8):
                            sl = pl.ds(k * 16, 16)
                            _vb[r, sl] = _vb[r, sl] * m
                    return cc

                lax.fori_loop(0, 8, rbody, 0)
                pltpu.async_copy(vals[b], acc.at[gi.at[j]], sem_s, add=True)
            for b in (0, 1):
                pltpu.make_async_copy(t2h.at[pl.ds(0, 128)],
                                      vals[b], sem_s).wait()
            return carry

        lax.fori_loop(0, 25, group, 0)
        plsc.subcore_barrier()
        pltpu.sync_copy(_ds(acc, s * CROWS, CROWS),
                        outp.at[c].at[pl.ds(s * CROWS, CROWS)])

    @functools.partial(
        pl.kernel,
        mesh=mesh,
        out_type=jax.ShapeDtypeStruct((NC, POOL_PAD, 128), _f32),
        scratch_types=[
            pltpu.VMEM_SHARED((POOL_PAD, 128), _f32),
            pltpu.VMEM((16, 128), jnp.int32),
            pltpu.VMEM((128, 128), _f32),
            pltpu.VMEM((128, 128), _f32),
            pltpu.SemaphoreType.DMA,
        ],
    )
    def sc_pool(a2, batchE, zpool, poolp, acc, idx_v, vals0, vals1, sem_s):
        c = lax.axis_index("c")
        s = lax.axis_index("s")
        w = s * NC + c
        vals = [vals0, vals1]
        pltpu.sync_copy(_ds(zpool, s * CROWS, CROWS), _ds(acc, s * CROWS, CROWS))
        plsc.subcore_barrier()

        @pl.when(w < 26)
        def _():
            pltpu.sync_copy(_ds(batchE, w * 16, 16), idx_v)
            for k in range(16):
                b = k & 1
                if k >= 2:
                    pltpu.make_async_copy(_ds(a2, 0, 128), vals[b], sem_s).wait()
                pltpu.sync_copy(_ds(a2, w * 2048 + k * 128, 128), vals[b])
                pltpu.async_copy(vals[b], acc.at[idx_v.at[k]], sem_s, add=True)
            for k in (14, 15):
                pltpu.make_async_copy(_ds(a2, 0, 128), vals[k & 1], sem_s).wait()
        plsc.subcore_barrier()
        pltpu.sync_copy(_ds(acc, s * CROWS, CROWS),
                        poolp.at[c].at[pl.ds(s * CROWS, CROWS)])

    return sc_hist, sc_s0, sc_agg1, sc_agg2g, sc_pool


def _prep_body(degp_ref, x_ref, mask_ref, dinv_ref, t0_ref):
    deg = degp_ref[0] + degp_ref[1] + 1.0
    dinv = lax.rsqrt(deg)
    dinv_ref[...] = dinv
    t0_ref[...] = dinv * x_ref[...] * mask_ref[...]


def _l1_body(s0p_ref, t0_ref, dinv_ref, mask_ref, W1_ref, b1_ref,
             t1a_ref, t1b_ref, t1c_ref):
    s0 = s0p_ref[0] + s0p_ref[1]
    dinv = dinv_ref[...]
    a0 = dinv * (s0 + t0_ref[...])
    h1 = jnp.maximum(a0 * W1_ref[...] + b1_ref[...], 0.0)
    t1 = jnp.where(mask_ref[...] > 0.0, dinv * h1, 0.0)
    t1a_ref[...] = t1
    t1b_ref[...] = jnp.concatenate([t1[:, 64:], t1[:, :64]], axis=1)
    t1c_ref[...] = t1[:, :64]


def _l2_body(s1_ref, t1c_ref, dinv_ref, mask_ref, W2_ref, b2_ref,
             t2_ref, u2_ref):
    dinv = dinv_ref[...]
    a1 = dinv * (s1_ref[...] + t1c_ref[...])
    h2 = jnp.dot(a1, W2_ref[...], preferred_element_type=_f32)
    h2 = jnp.maximum(h2 + b2_ref[...], 0.0)
    t2 = jnp.where(mask_ref[...] > 0.0, dinv * h2, 0.0)
    t2_ref[...] = t2
    u2_ref[...] = dinv * t2


def _idx_body(srcE_ref, dstE_ref, gsrc_ref, idxp_ref):
    d = dstE_ref[...]
    gsrc_ref[...] = srcE_ref[...] + NPAD * (d & 1)
    for q in range(NQ1):
        lo = q * QR1
        loc = jnp.where((d >= lo) & (d < lo + QR1), (d - lo) >> 1, HQ)
        idxp_ref[q, :, :] = loc


def _head_body(poolp_ref, e2p_ref, cntp_ref, W3_ref, b3_ref, f1W_ref, f1b_ref,
               f2W_ref, f2b_ref, out_ref):
    pool = (poolp_ref[0] + poolp_ref[1]) + (e2p_ref[0] + e2p_ref[1])
    cnt = cntp_ref[0] + cntp_ref[1]
    g = pool / jnp.maximum(cnt, 1.0)
    G = jnp.dot(g, W3_ref[...], preferred_element_type=_f32) + b3_ref[...]
    G = jnp.maximum(jnp.dot(G, f1W_ref[...], preferred_element_type=_f32)
                    + f1b_ref[...], 0.0)
    out_ref[...] = jnp.dot(G, f2W_ref[...], preferred_element_type=_f32) + f2b_ref[...]


def _node_spec(blk, ncols):
    return pl.BlockSpec((blk, ncols), lambda i: (i, 0))


def kernel(x, edge_index, batch, W1, b1, W2, b2, W3, b3, fc1_W, fc1_b, fc2_W, fc2_b):
    sc_hist, sc_s0, sc_agg1, sc_agg2g, sc_pool = _build_sc_kernels()
    src = edge_index[0].astype(jnp.int32)
    dst = edge_index[1].astype(jnp.int32)
    batch = batch.astype(jnp.int32)

    epad = 50000 + (jnp.arange(NE_P - NE, dtype=jnp.int32) % (NPAD - N))
    srcE = jnp.concatenate([src, epad]).reshape(EROWS, 128)
    dstE = jnp.concatenate([dst, epad]).reshape(EROWS, 128)
    bpad = NG + (jnp.arange(NPAD - N, dtype=jnp.int32) % 32)
    batch_flat = jnp.concatenate([batch, bpad])
    batchE = batch_flat.reshape(NROWS, 128)
    x_flat = jnp.concatenate([x[:, 0], jnp.zeros((NPAD - N,), _f32)])
    mask_flat = (jnp.arange(NPAD) < N).astype(_f32)

    ones_hbm = jnp.ones((16, 128), _f32)
    z1 = jnp.zeros((NPAD,), _f32)
    zc = jnp.zeros((POOL_PAD,), _f32)
    z128 = jnp.zeros((NPAD, 128), _f32)
    zpool = jnp.zeros((POOL_PAD, 128), _f32)

    degp, cntp = sc_hist(dstE, batchE, ones_hbm, z1, zc)

    E_BLK = 320
    gsrc, idxp = pl.pallas_call(
        _idx_body,
        grid=(EROWS // E_BLK,),
        in_specs=[pl.BlockSpec((E_BLK, 128), lambda i: (i, 0))] * 2,
        out_specs=[pl.BlockSpec((E_BLK, 128), lambda i: (i, 0)),
                   pl.BlockSpec((NQ1, E_BLK, 128), lambda i: (0, i, 0))],
        out_shape=[jax.ShapeDtypeStruct((EROWS, 128), jnp.int32),
                   jax.ShapeDtypeStruct((NQ1, EROWS, 128), jnp.int32)],
    )(srcE, dstE)
    idxpE = idxp.reshape(NQ1 * EROWS, 128)

    dinv2, t02 = pl.pallas_call(
        _prep_body,
        out_shape=[jax.ShapeDtypeStruct((NROWS, 128), _f32)] * 2,
    )(degp.reshape(NC, NROWS, 128), x_flat.reshape(NROWS, 128),
      mask_flat.reshape(NROWS, 128))
    dinvc = dinv2.reshape(NPAD, 1)
    dinv_flat = dinv2.reshape(NPAD)
    maskc = mask_flat.reshape(NPAD, 1)
    t0_flat = t02.reshape(NPAD)

    s0p = sc_s0(srcE, dstE, t0_flat, z1)

    BLK = 512
    grid = (NPAD // BLK,)
    W1p = jnp.zeros((1, 128), _f32).at[0, :64].set(W1[0])
    b1p = jnp.zeros((1, 128), _f32).at[0, :64].set(b1)
    t1a, t1b, t1c = pl.pallas_call(
        _l1_body,
        grid=grid,
        in_specs=[
            pl.BlockSpec((NC, BLK, 1), lambda i: (0, i, 0)),
            _node_spec(BLK, 1), _node_spec(BLK, 1), _node_spec(BLK, 1),
            pl.BlockSpec((1, 128), lambda i: (0, 0)),
            pl.BlockSpec((1, 128), lambda i: (0, 0)),
        ],
        out_specs=[_node_spec(BLK, 128), _node_spec(BLK, 128),
                   _node_spec(BLK, 64)],
        out_shape=[jax.ShapeDtypeStruct((NPAD, 128), _f32),
                   jax.ShapeDtypeStruct((NPAD, 128), _f32),
                   jax.ShapeDtypeStruct((NPAD, 64), _f32)],
    )(s0p.reshape(NC, NPAD, 1), t0_flat.reshape(NPAD, 1), dinvc, maskc,
      W1p, b1p)
    Q = jnp.concatenate([t1a, t1b], axis=0)

    s1p = sc_agg1(gsrc, idxpE, Q, z128)
    s1_64 = s1p.reshape(NPAD, 64)

    t2, u2 = pl.pallas_call(
        _l2_body,
        grid=grid,
        in_specs=[
            _node_spec(BLK, 64), _node_spec(BLK, 64),
            _node_spec(BLK, 1), _node_spec(BLK, 1),
            pl.BlockSpec((64, 128), lambda i: (0, 0)),
            pl.BlockSpec((1, 128), lambda i: (0, 0)),
        ],
        out_specs=[_node_spec(BLK, 128)] * 2,
        out_shape=[jax.ShapeDtypeStruct((NPAD, 128), _f32)] * 2,
    )(s1_64, t1c, dinvc, maskc, W2, b2.reshape(1, 128))

    e2p = sc_agg2g(srcE, dstE, t2, dinv_flat, batch_flat, zpool)

    poolp = sc_pool(u2, batchE, zpool)

    f2Wp = jnp.zeros((128, 128), _f32).at[:, :12].set(fc2_W)
    f2bp = jnp.zeros((1, 128), _f32).at[0, :12].set(fc2_b)
    out_p = pl.pallas_call(
        _head_body,
        out_shape=jax.ShapeDtypeStruct((NG, 128), _f32),
    )(poolp[:, :NG, :], e2p[:, :NG, :], cntp[:, :NG].reshape(NC, NG, 1), W3,
      b3.reshape(1, 256), fc1_W, fc1_b.reshape(1, 128), f2Wp, f2bp)
    return out_p[:, :12]
